# baseline (speedup 1.0000x reference)
import jax
import jax.numpy as jnp
from jax import lax
from jax.experimental import pallas as pl
from jax.experimental.pallas import tpu as pltpu
from jax.experimental.pallas import tpu_sc as plsc

N_NODES = 10000
D = 128
E = 320000
NC = 2
NS = 16
NW = NC * NS
E_PER_W = E // NW
EC = 80
NCHUNK = E_PER_W // EC
NG = EC // 16


def _decoder_body(z_hbm, col_hbm, row_hbm, out_hbm,
                  cols, rows, zc0, zr0, zc1, zr1, outv, sem0, sem1):
    cid = lax.axis_index("c")
    sid = lax.axis_index("s")
    wid = sid * NC + cid
    base = wid * E_PER_W
    lane = lax.iota(jnp.int32, 16)

    pltpu.sync_copy(col_hbm.at[pl.ds(base, E_PER_W)], cols)
    pltpu.sync_copy(row_hbm.at[pl.ds(base, E_PER_W)], rows)

    def fetch(c, zc, zr, sem):
        off = c * EC
        h0 = pltpu.async_copy(z_hbm.at[cols.at[pl.ds(off, EC)]], zc, sem)
        h1 = pltpu.async_copy(z_hbm.at[rows.at[pl.ds(off, EC)]], zr, sem)
        return h0, h1

    def compute(c, zc, zr):
        off = base + c * EC

        def group(g, carry2):
            eids = g * 16 + lane
            acc0 = jnp.zeros((16,), jnp.float32)
            acc1 = jnp.zeros((16,), jnp.float32)
            for d in range(0, D, 2):
                d0 = jnp.full((16,), d, jnp.int32)
                d1 = jnp.full((16,), d + 1, jnp.int32)
                acc0 = acc0 + plsc.load_gather(zc, [eids, d0]) * plsc.load_gather(zr, [eids, d0])
                acc1 = acc1 + plsc.load_gather(zc, [eids, d1]) * plsc.load_gather(zr, [eids, d1])
            dot = acc0 + acc1
            sig = 1.0 / (1.0 + jnp.exp(-dot))
            outv[pl.ds(g * 16, 16)] = sig
            return carry2

        lax.fori_loop(0, NG, group, 0)
        pltpu.sync_copy(outv, out_hbm.at[pl.ds(off, EC)])

    p0, p1 = fetch(0, zc0, zr0, sem0)
    p0.wait()
    p1.wait()

    def step(i, carry):
        c = i * 2
        h0, h1 = fetch(c + 1, zc1, zr1, sem1)
        compute(c, zc0, zr0)
        h0.wait()
        h1.wait()
        g0, g1 = fetch(c + 2, zc0, zr0, sem0)
        compute(c + 1, zc1, zr1)
        g0.wait()
        g1.wait()
        return carry

    lax.fori_loop(0, (NCHUNK - 1) // 2, step, 0)
    compute(NCHUNK - 1, zc0, zr0)


def kernel(z, edge_index):
    ei = edge_index.astype(jnp.int32)
    col = ei[0]
    row = ei[1]
    mesh = plsc.VectorSubcoreMesh(core_axis_name="c", subcore_axis_name="s")
    f = pl.kernel(
        _decoder_body,
        mesh=mesh,
        out_type=jax.ShapeDtypeStruct((E,), jnp.float32),
        compiler_params=pltpu.CompilerParams(needs_layout_passes=False),
        scratch_types=[
            pltpu.VMEM((E_PER_W,), jnp.int32),
            pltpu.VMEM((E_PER_W,), jnp.int32),
            pltpu.VMEM((EC, D), jnp.float32),
            pltpu.VMEM((EC, D), jnp.float32),
            pltpu.VMEM((EC, D), jnp.float32),
            pltpu.VMEM((EC, D), jnp.float32),
            pltpu.VMEM((EC,), jnp.float32),
            pltpu.SemaphoreType.DMA,
            pltpu.SemaphoreType.DMA,
        ],
    )
    return f(z, col, row)


# speedup vs baseline: 5.1767x; 5.1767x over previous
import jax
import jax.numpy as jnp
from jax import lax
from jax.experimental import pallas as pl
from jax.experimental.pallas import tpu as pltpu
from jax.experimental.pallas import tpu_sc as plsc

N_NODES = 10000
D = 128
E = 320000
NC = 2
NS = 16
NW = NC * NS
E_PER_W = E // NW
EC = 80
NCHUNK = E_PER_W // EC
NG = EC // 16


def _decoder_body(z_hbm, col_hbm, row_hbm, out_hbm,
                  cols, rows, zc0, zr0, zc1, zr1, outv, pscr, sem0, sem1):
    cid = lax.axis_index("c")
    sid = lax.axis_index("s")
    wid = sid * NC + cid
    base = wid * E_PER_W
    lane = lax.iota(jnp.int32, 16)
    lane17 = lane * 17

    pltpu.sync_copy(col_hbm.at[pl.ds(base, E_PER_W)], cols)
    pltpu.sync_copy(row_hbm.at[pl.ds(base, E_PER_W)], rows)

    def fetch(c, zc, zr, sem):
        off = c * EC
        h0 = pltpu.async_copy(z_hbm.at[cols.at[pl.ds(off, EC)]], zc, sem)
        h1 = pltpu.async_copy(z_hbm.at[rows.at[pl.ds(off, EC)]], zr, sem)
        return h0, h1

    def compute(c, zc, zr):
        off = base + c * EC

        def group(g, carry2):
            e0 = g * 16
            for j in range(16):
                e = e0 + j
                acc0 = zc[e, pl.ds(0, 16)] * zr[e, pl.ds(0, 16)]
                acc1 = zc[e, pl.ds(16, 16)] * zr[e, pl.ds(16, 16)]
                for k in range(2, D // 16, 2):
                    acc0 = acc0 + zc[e, pl.ds(k * 16, 16)] * zr[e, pl.ds(k * 16, 16)]
                    acc1 = acc1 + zc[e, pl.ds(k * 16 + 16, 16)] * zr[e, pl.ds(k * 16 + 16, 16)]
                pscr[pl.ds(j * 17, 16)] = acc0 + acc1
            t = [plsc.load_gather(pscr, [lane17 + l]) for l in range(16)]
            while len(t) > 1:
                t = [t[i] + t[i + 1] for i in range(0, len(t), 2)]
            sig = 1.0 / (1.0 + jnp.exp(-t[0]))
            outv[pl.ds(e0, 16)] = sig
            return carry2

        lax.fori_loop(0, NG, group, 0)
        pltpu.sync_copy(outv, out_hbm.at[pl.ds(off, EC)])

    p0, p1 = fetch(0, zc0, zr0, sem0)
    p0.wait()
    p1.wait()

    def step(i, carry):
        c = i * 2
        h0, h1 = fetch(c + 1, zc1, zr1, sem1)
        compute(c, zc0, zr0)
        h0.wait()
        h1.wait()
        g0, g1 = fetch(c + 2, zc0, zr0, sem0)
        compute(c + 1, zc1, zr1)
        g0.wait()
        g1.wait()
        return carry

    lax.fori_loop(0, (NCHUNK - 1) // 2, step, 0)
    compute(NCHUNK - 1, zc0, zr0)


def kernel(z, edge_index):
    ei = edge_index.astype(jnp.int32)
    col = ei[0]
    row = ei[1]
    mesh = plsc.VectorSubcoreMesh(core_axis_name="c", subcore_axis_name="s")
    f = pl.kernel(
        _decoder_body,
        mesh=mesh,
        out_type=jax.ShapeDtypeStruct((E,), jnp.float32),
        compiler_params=pltpu.CompilerParams(needs_layout_passes=False),
        scratch_types=[
            pltpu.VMEM((E_PER_W,), jnp.int32),
            pltpu.VMEM((E_PER_W,), jnp.int32),
            pltpu.VMEM((EC, D), jnp.float32),
            pltpu.VMEM((EC, D), jnp.float32),
            pltpu.VMEM((EC, D), jnp.float32),
            pltpu.VMEM((EC, D), jnp.float32),
            pltpu.VMEM((EC,), jnp.float32),
            pltpu.VMEM((16 * 17,), jnp.float32),
            pltpu.SemaphoreType.DMA,
            pltpu.SemaphoreType.DMA,
        ],
    )
    return f(z, col, row)
